# Initial kernel scaffold; baseline (speedup 1.0000x reference)
import jax
import jax.numpy as jnp
from jax.experimental import pallas as pl


def kernel(x, edge_index, W, b):
    raise NotImplementedError("write your pallas kernel here")


# speedup vs baseline: 12.6291x; 12.6291x over previous
import functools

import jax
import jax.numpy as jnp
from jax import lax
from jax.experimental import pallas as pl
from jax.experimental.pallas import tpu as pltpu
from jax.experimental.pallas import tpu_sc as plsc

N = 10000
E = 320000
IN_F = 128
OUT_F = 16

NC = 2
NS = 16
NW = NC * NS
L = 16

N_PAD = 10240
ROWS_SC_TILE = N_PAD // NS
ROWS_W_TILE = N_PAD // NW
FLAT = N_PAD * OUT_F
FLAT_TILE = FLAT // NW

CPT = 80
N_CHUNKS = NW * CPT
E_PAD = N_CHUNKS * 128

_mesh = plsc.VectorSubcoreMesh(core_axis_name="c", subcore_axis_name="s")
_sc_params = pltpu.CompilerParams(use_tc_tiling_on_sc=False)
f32 = jnp.float32
i32 = jnp.int32


def _wid():
    return lax.axis_index("s") * NC + lax.axis_index("c")


def _round_body_deg(y_hbm, src_hbm, dst_hbm, ones_hbm, zeros_hbm,
                    p_hbm, da_hbm, db_hbm,
                    srcv, dstv, onesv, rowsv, stagev, accum, dega, degb, sem):
    cid = lax.axis_index("c")
    sid = lax.axis_index("s")
    wid = sid * NC + cid
    r0 = sid * ROWS_SC_TILE
    pltpu.sync_copy(zeros_hbm, stagev)
    pltpu.sync_copy(stagev, accum.at[pl.ds(r0, ROWS_SC_TILE)])
    pltpu.sync_copy(stagev, dega.at[pl.ds(r0, ROWS_SC_TILE)])
    pltpu.sync_copy(stagev, degb.at[pl.ds(r0, ROWS_SC_TILE)])
    pltpu.sync_copy(ones_hbm, onesv)
    c0 = wid * CPT
    pltpu.sync_copy(src_hbm.at[pl.ds(c0, CPT)], srcv)
    pltpu.sync_copy(dst_hbm.at[pl.ds(c0, CPT)], dstv)
    plsc.subcore_barrier()

    def body(j, c):
        pltpu.async_copy(y_hbm.at[srcv.at[j]], rowsv, sem).wait()
        pltpu.sync_copy(rowsv, accum.at[dstv.at[j]], add=True)
        pltpu.sync_copy(onesv, dega.at[srcv.at[j]], add=True)
        pltpu.sync_copy(onesv, degb.at[dstv.at[j]], add=True)
        return c

    lax.fori_loop(0, CPT, body, 0)
    plsc.subcore_barrier()
    pltpu.sync_copy(accum.at[pl.ds(r0, ROWS_SC_TILE)], stagev)
    pltpu.sync_copy(stagev, p_hbm.at[cid].at[pl.ds(r0, ROWS_SC_TILE)])
    pltpu.sync_copy(dega.at[pl.ds(r0, ROWS_SC_TILE)], stagev)
    pltpu.sync_copy(stagev, da_hbm.at[cid].at[pl.ds(r0, ROWS_SC_TILE)])
    pltpu.sync_copy(degb.at[pl.ds(r0, ROWS_SC_TILE)], stagev)
    pltpu.sync_copy(stagev, db_hbm.at[cid].at[pl.ds(r0, ROWS_SC_TILE)])


def _round_body(y_hbm, src_hbm, dst_hbm, zeros_hbm, p_hbm,
                srcv, dstv, rowsv, stagev, accum, sem):
    cid = lax.axis_index("c")
    sid = lax.axis_index("s")
    wid = sid * NC + cid
    r0 = sid * ROWS_SC_TILE
    pltpu.sync_copy(zeros_hbm, stagev)
    pltpu.sync_copy(stagev, accum.at[pl.ds(r0, ROWS_SC_TILE)])
    c0 = wid * CPT
    pltpu.sync_copy(src_hbm.at[pl.ds(c0, CPT)], srcv)
    pltpu.sync_copy(dst_hbm.at[pl.ds(c0, CPT)], dstv)
    plsc.subcore_barrier()

    def body(j, c):
        pltpu.async_copy(y_hbm.at[srcv.at[j]], rowsv, sem).wait()
        pltpu.sync_copy(rowsv, accum.at[dstv.at[j]], add=True)
        return c

    lax.fori_loop(0, CPT, body, 0)
    plsc.subcore_barrier()
    pltpu.sync_copy(accum.at[pl.ds(r0, ROWS_SC_TILE)], stagev)
    pltpu.sync_copy(stagev, p_hbm.at[cid].at[pl.ds(r0, ROWS_SC_TILE)])


def _round_deg(y_p, src2, dst2, ones_in, zeros_in):
    kern = pl.kernel(
        _round_body_deg,
        out_type=[jax.ShapeDtypeStruct((NC, N_PAD, OUT_F), f32)] * 3,
        mesh=_mesh,
        compiler_params=_sc_params,
        scratch_types=[
            pltpu.VMEM((CPT, 128), i32),
            pltpu.VMEM((CPT, 128), i32),
            pltpu.VMEM((128, OUT_F), f32),
            pltpu.VMEM((128, OUT_F), f32),
            pltpu.VMEM((ROWS_SC_TILE, OUT_F), f32),
            pltpu.VMEM_SHARED((N_PAD, OUT_F), f32),
            pltpu.VMEM_SHARED((N_PAD, OUT_F), f32),
            pltpu.VMEM_SHARED((N_PAD, OUT_F), f32),
            pltpu.SemaphoreType.DMA,
        ],
    )
    return kern(y_p, src2, dst2, ones_in, zeros_in)


def _round(t_in, src2, dst2, zeros_in):
    kern = pl.kernel(
        _round_body,
        out_type=jax.ShapeDtypeStruct((NC, N_PAD, OUT_F), f32),
        mesh=_mesh,
        compiler_params=_sc_params,
        scratch_types=[
            pltpu.VMEM((CPT, 128), i32),
            pltpu.VMEM((CPT, 128), i32),
            pltpu.VMEM((128, OUT_F), f32),
            pltpu.VMEM((ROWS_SC_TILE, OUT_F), f32),
            pltpu.VMEM_SHARED((N_PAD, OUT_F), f32),
            pltpu.SemaphoreType.DMA,
        ],
    )
    return kern(t_in, src2, dst2, zeros_in)


def _nrsqrt(dd):
    ii = lax.bitcast_convert_type(dd, i32)
    ii = 0x5F3759DF - (ii >> 1)
    yv = lax.bitcast_convert_type(ii, f32)
    yv = yv * (1.5 - 0.5 * dd * yv * yv)
    yv = yv * (1.5 - 0.5 * dd * yv * yv)
    yv = yv * (1.5 - 0.5 * dd * yv * yv)
    return yv


def _combine_norm_body(p_hbm, da_hbm, db_hbm, t1_hbm, on_hbm, in_hbm, av, bv, ov):
    g0 = _wid() * FLAT_TILE

    def _load2(src):
        pltpu.sync_copy(src.at[0].at[pl.ds(g0, FLAT_TILE)], av)
        pltpu.sync_copy(src.at[1].at[pl.ds(g0, FLAT_TILE)], bv)

    _load2(p_hbm)

    def add_body(i, c):
        s = pl.ds(i * L, L)
        ov[s] = av[s] + bv[s]
        return c

    lax.fori_loop(0, ROWS_W_TILE, add_body, 0)
    pltpu.sync_copy(ov, t1_hbm.at[pl.ds(g0, FLAT_TILE)])

    def norm_body(i, c):
        s = pl.ds(i * L, L)
        ov[s] = _nrsqrt(jnp.maximum(av[s] + bv[s], 1.0))
        return c

    _load2(da_hbm)
    lax.fori_loop(0, ROWS_W_TILE, norm_body, 0)
    pltpu.sync_copy(ov, on_hbm.at[pl.ds(g0, FLAT_TILE)])

    _load2(db_hbm)
    lax.fori_loop(0, ROWS_W_TILE, norm_body, 0)
    pltpu.sync_copy(ov, in_hbm.at[pl.ds(g0, FLAT_TILE)])


def _combine_norm(p1, dA, dB):
    kern = pl.kernel(
        _combine_norm_body,
        out_type=[jax.ShapeDtypeStruct((FLAT,), f32)] * 3,
        mesh=_mesh,
        compiler_params=_sc_params,
        scratch_types=[pltpu.VMEM((FLAT_TILE,), f32)] * 3,
    )
    return kern(p1, dA, dB)


def _combine_scale_body(p_hbm, n_hbm, o_hbm, av, bv, cv):
    g0 = _wid() * FLAT_TILE
    pltpu.sync_copy(p_hbm.at[0].at[pl.ds(g0, FLAT_TILE)], av)
    pltpu.sync_copy(p_hbm.at[1].at[pl.ds(g0, FLAT_TILE)], bv)
    pltpu.sync_copy(n_hbm.at[pl.ds(g0, FLAT_TILE)], cv)

    def body(i, c):
        s = pl.ds(i * L, L)
        av[s] = (av[s] + bv[s]) * cv[s]
        return c

    lax.fori_loop(0, ROWS_W_TILE, body, 0)
    pltpu.sync_copy(av, o_hbm.at[pl.ds(g0, FLAT_TILE)])


def _combine_scale(p2, onorm):
    kern = pl.kernel(
        _combine_scale_body,
        out_type=jax.ShapeDtypeStruct((FLAT,), f32),
        mesh=_mesh,
        compiler_params=_sc_params,
        scratch_types=[pltpu.VMEM((FLAT_TILE,), f32)] * 3,
    )
    return kern(p2, onorm)


def _combine_scale_bias_body(p_hbm, n_hbm, b_hbm, o_hbm, av, bv, cv, bvv):
    g0 = _wid() * FLAT_TILE
    pltpu.sync_copy(p_hbm.at[0].at[pl.ds(g0, FLAT_TILE)], av)
    pltpu.sync_copy(p_hbm.at[1].at[pl.ds(g0, FLAT_TILE)], bv)
    pltpu.sync_copy(n_hbm.at[pl.ds(g0, FLAT_TILE)], cv)
    pltpu.sync_copy(b_hbm, bvv)
    bvec = bvv[...]

    def body(i, c):
        s = pl.ds(i * L, L)
        av[s] = (av[s] + bv[s]) * cv[s] + bvec
        return c

    lax.fori_loop(0, ROWS_W_TILE, body, 0)
    pltpu.sync_copy(av, o_hbm.at[pl.ds(g0, FLAT_TILE)])


def _combine_scale_bias(p3, inorm, b):
    kern = pl.kernel(
        _combine_scale_bias_body,
        out_type=jax.ShapeDtypeStruct((FLAT,), f32),
        mesh=_mesh,
        compiler_params=_sc_params,
        scratch_types=[
            pltpu.VMEM((FLAT_TILE,), f32),
            pltpu.VMEM((FLAT_TILE,), f32),
            pltpu.VMEM((FLAT_TILE,), f32),
            pltpu.VMEM((OUT_F,), f32),
        ],
    )
    return kern(p3, inorm, b)


def _mm_body(x_ref, w_ref, o_ref):
    o_ref[...] = jnp.dot(x_ref[...], w_ref[...], preferred_element_type=f32)


def _matmul(x, W):
    return pl.pallas_call(
        _mm_body,
        grid=(10,),
        in_specs=[
            pl.BlockSpec((N // 10, IN_F), lambda i: (i, 0)),
            pl.BlockSpec((IN_F, OUT_F), lambda i: (0, 0)),
        ],
        out_specs=pl.BlockSpec((N // 10, OUT_F), lambda i: (i, 0)),
        out_shape=jax.ShapeDtypeStruct((N, OUT_F), f32),
    )(x, W)


def kernel(x, edge_index, W, b):
    src = edge_index[0].astype(i32)
    dst = edge_index[1].astype(i32)
    pad = N + (jnp.arange(E_PAD - E, dtype=i32) % (N_PAD - N))
    src2 = jnp.concatenate([src, pad]).reshape(N_CHUNKS, 128)
    dst2 = jnp.concatenate([dst, pad]).reshape(N_CHUNKS, 128)

    y = _matmul(x, W)
    y_p = jnp.concatenate([y, jnp.zeros((N_PAD - N, OUT_F), f32)], axis=0)

    ones_in = jnp.ones((128, OUT_F), f32)
    zeros_in = jnp.zeros((ROWS_SC_TILE, OUT_F), f32)

    p1, dA, dB = _round_deg(y_p, src2, dst2, ones_in, zeros_in)
    t1f, onormf, inormf = _combine_norm(
        p1.reshape(NC, FLAT), dA.reshape(NC, FLAT), dB.reshape(NC, FLAT))

    p2 = _round(t1f.reshape(N_PAD, OUT_F), src2, dst2, zeros_in)
    t2f = _combine_scale(p2.reshape(NC, FLAT), onormf)

    p3 = _round(t2f.reshape(N_PAD, OUT_F), src2, dst2, zeros_in)
    rstf = _combine_scale_bias(p3.reshape(NC, FLAT), inormf, b)

    return rstf.reshape(N_PAD, OUT_F)[:N]


# speedup vs baseline: 23.0423x; 1.8245x over previous
import functools

import jax
import jax.numpy as jnp
from jax import lax
from jax.experimental import pallas as pl
from jax.experimental.pallas import tpu as pltpu
from jax.experimental.pallas import tpu_sc as plsc

N = 10000
E = 320000
IN_F = 128
OUT_F = 16

NC = 2
NS = 16
NW = NC * NS
L = 16

N_PAD = 10240
ROWS_SC_TILE = N_PAD // NS
ROWS_W_TILE = N_PAD // NW
FLAT = N_PAD * OUT_F
FLAT_TILE = FLAT // NW

CPT = 80
N_CHUNKS = NW * CPT
E_PAD = N_CHUNKS * 128

_mesh = plsc.VectorSubcoreMesh(core_axis_name="c", subcore_axis_name="s")
_sc_params = pltpu.CompilerParams(use_tc_tiling_on_sc=False)
f32 = jnp.float32
i32 = jnp.int32


def _wid():
    return lax.axis_index("s") * NC + lax.axis_index("c")


NBUF = 4


def _gather_ring(y_hbm, srcv, rowsv, sems, consume):
    def _buf(b):
        return rowsv.at[pl.ds(b * 128, 128)]

    def _gstart(j, b):
        return pltpu.async_copy(y_hbm.at[srcv.at[j]], _buf(b), sems[b])

    for b in range(NBUF):
        _gstart(b, b)

    def outer(q, c):
        for b in range(NBUF):
            j = q * NBUF + b
            pltpu.make_async_copy(y_hbm.at[srcv.at[j]], _buf(b), sems[b]).wait()
            consume(j, _buf(b))
            jn = j + NBUF

            @pl.when(jn < CPT)
            def _():
                _gstart(jn, b)
        return c

    lax.fori_loop(0, CPT // NBUF, outer, 0)


def _round_body_deg(y_hbm, src_hbm, dst_hbm, ones_hbm, zeros_hbm,
                    p_hbm, da_hbm, db_hbm,
                    srcv, dstv, onesv, rowsv, stagev, accum, dega, degb,
                    s0, s1, s2, s3):
    cid = lax.axis_index("c")
    sid = lax.axis_index("s")
    wid = sid * NC + cid
    r0 = sid * ROWS_SC_TILE
    pltpu.sync_copy(zeros_hbm, stagev)
    pltpu.sync_copy(stagev, accum.at[pl.ds(r0, ROWS_SC_TILE)])
    pltpu.sync_copy(stagev, dega.at[pl.ds(r0, ROWS_SC_TILE)])
    pltpu.sync_copy(stagev, degb.at[pl.ds(r0, ROWS_SC_TILE)])
    pltpu.sync_copy(ones_hbm, onesv)
    c0 = wid * CPT
    pltpu.sync_copy(src_hbm.at[pl.ds(c0, CPT)], srcv)
    pltpu.sync_copy(dst_hbm.at[pl.ds(c0, CPT)], dstv)
    plsc.subcore_barrier()

    def consume(j, rows):
        pltpu.sync_copy(rows, accum.at[dstv.at[j]], add=True)
        pltpu.sync_copy(onesv, dega.at[srcv.at[j]], add=True)
        pltpu.sync_copy(onesv, degb.at[dstv.at[j]], add=True)

    _gather_ring(y_hbm, srcv, rowsv, (s0, s1, s2, s3), consume)
    plsc.subcore_barrier()
    pltpu.sync_copy(accum.at[pl.ds(r0, ROWS_SC_TILE)], stagev)
    pltpu.sync_copy(stagev, p_hbm.at[cid].at[pl.ds(r0, ROWS_SC_TILE)])
    pltpu.sync_copy(dega.at[pl.ds(r0, ROWS_SC_TILE)], stagev)
    pltpu.sync_copy(stagev, da_hbm.at[cid].at[pl.ds(r0, ROWS_SC_TILE)])
    pltpu.sync_copy(degb.at[pl.ds(r0, ROWS_SC_TILE)], stagev)
    pltpu.sync_copy(stagev, db_hbm.at[cid].at[pl.ds(r0, ROWS_SC_TILE)])


def _round_body(y_hbm, src_hbm, dst_hbm, zeros_hbm, p_hbm,
                srcv, dstv, rowsv, stagev, accum, s0, s1, s2, s3):
    cid = lax.axis_index("c")
    sid = lax.axis_index("s")
    wid = sid * NC + cid
    r0 = sid * ROWS_SC_TILE
    pltpu.sync_copy(zeros_hbm, stagev)
    pltpu.sync_copy(stagev, accum.at[pl.ds(r0, ROWS_SC_TILE)])
    c0 = wid * CPT
    pltpu.sync_copy(src_hbm.at[pl.ds(c0, CPT)], srcv)
    pltpu.sync_copy(dst_hbm.at[pl.ds(c0, CPT)], dstv)
    plsc.subcore_barrier()

    def consume(j, rows):
        pltpu.sync_copy(rows, accum.at[dstv.at[j]], add=True)

    _gather_ring(y_hbm, srcv, rowsv, (s0, s1, s2, s3), consume)
    plsc.subcore_barrier()
    pltpu.sync_copy(accum.at[pl.ds(r0, ROWS_SC_TILE)], stagev)
    pltpu.sync_copy(stagev, p_hbm.at[cid].at[pl.ds(r0, ROWS_SC_TILE)])


def _round_deg(y_p, src2, dst2, ones_in, zeros_in):
    kern = pl.kernel(
        _round_body_deg,
        out_type=[jax.ShapeDtypeStruct((NC, N_PAD, OUT_F), f32)] * 3,
        mesh=_mesh,
        compiler_params=_sc_params,
        scratch_types=[
            pltpu.VMEM((CPT, 128), i32),
            pltpu.VMEM((CPT, 128), i32),
            pltpu.VMEM((128, OUT_F), f32),
            pltpu.VMEM((NBUF * 128, OUT_F), f32),
            pltpu.VMEM((ROWS_SC_TILE, OUT_F), f32),
            pltpu.VMEM_SHARED((N_PAD, OUT_F), f32),
            pltpu.VMEM_SHARED((N_PAD, OUT_F), f32),
            pltpu.VMEM_SHARED((N_PAD, OUT_F), f32),
            pltpu.SemaphoreType.DMA,
            pltpu.SemaphoreType.DMA,
            pltpu.SemaphoreType.DMA,
            pltpu.SemaphoreType.DMA,
        ],
    )
    return kern(y_p, src2, dst2, ones_in, zeros_in)


def _round(t_in, src2, dst2, zeros_in):
    kern = pl.kernel(
        _round_body,
        out_type=jax.ShapeDtypeStruct((NC, N_PAD, OUT_F), f32),
        mesh=_mesh,
        compiler_params=_sc_params,
        scratch_types=[
            pltpu.VMEM((CPT, 128), i32),
            pltpu.VMEM((CPT, 128), i32),
            pltpu.VMEM((NBUF * 128, OUT_F), f32),
            pltpu.VMEM((ROWS_SC_TILE, OUT_F), f32),
            pltpu.VMEM_SHARED((N_PAD, OUT_F), f32),
            pltpu.SemaphoreType.DMA,
            pltpu.SemaphoreType.DMA,
            pltpu.SemaphoreType.DMA,
            pltpu.SemaphoreType.DMA,
        ],
    )
    return kern(t_in, src2, dst2, zeros_in)


def _nrsqrt(dd):
    ii = lax.bitcast_convert_type(dd, i32)
    ii = 0x5F3759DF - (ii >> 1)
    yv = lax.bitcast_convert_type(ii, f32)
    yv = yv * (1.5 - 0.5 * dd * yv * yv)
    yv = yv * (1.5 - 0.5 * dd * yv * yv)
    yv = yv * (1.5 - 0.5 * dd * yv * yv)
    return yv


def _combine_norm_body(p_hbm, da_hbm, db_hbm, t1_hbm, on_hbm, in_hbm, av, bv, ov):
    g0 = _wid() * FLAT_TILE

    def _load2(src):
        pltpu.sync_copy(src.at[0].at[pl.ds(g0, FLAT_TILE)], av)
        pltpu.sync_copy(src.at[1].at[pl.ds(g0, FLAT_TILE)], bv)

    _load2(p_hbm)

    def add_body(i, c):
        s = pl.ds(i * L, L)
        ov[s] = av[s] + bv[s]
        return c

    lax.fori_loop(0, ROWS_W_TILE, add_body, 0)
    pltpu.sync_copy(ov, t1_hbm.at[pl.ds(g0, FLAT_TILE)])

    def norm_body(i, c):
        s = pl.ds(i * L, L)
        ov[s] = _nrsqrt(jnp.maximum(av[s] + bv[s], 1.0))
        return c

    _load2(da_hbm)
    lax.fori_loop(0, ROWS_W_TILE, norm_body, 0)
    pltpu.sync_copy(ov, on_hbm.at[pl.ds(g0, FLAT_TILE)])

    _load2(db_hbm)
    lax.fori_loop(0, ROWS_W_TILE, norm_body, 0)
    pltpu.sync_copy(ov, in_hbm.at[pl.ds(g0, FLAT_TILE)])


def _combine_norm(p1, dA, dB):
    kern = pl.kernel(
        _combine_norm_body,
        out_type=[jax.ShapeDtypeStruct((FLAT,), f32)] * 3,
        mesh=_mesh,
        compiler_params=_sc_params,
        scratch_types=[pltpu.VMEM((FLAT_TILE,), f32)] * 3,
    )
    return kern(p1, dA, dB)


def _combine_scale_body(p_hbm, n_hbm, o_hbm, av, bv, cv):
    g0 = _wid() * FLAT_TILE
    pltpu.sync_copy(p_hbm.at[0].at[pl.ds(g0, FLAT_TILE)], av)
    pltpu.sync_copy(p_hbm.at[1].at[pl.ds(g0, FLAT_TILE)], bv)
    pltpu.sync_copy(n_hbm.at[pl.ds(g0, FLAT_TILE)], cv)

    def body(i, c):
        s = pl.ds(i * L, L)
        av[s] = (av[s] + bv[s]) * cv[s]
        return c

    lax.fori_loop(0, ROWS_W_TILE, body, 0)
    pltpu.sync_copy(av, o_hbm.at[pl.ds(g0, FLAT_TILE)])


def _combine_scale(p2, onorm):
    kern = pl.kernel(
        _combine_scale_body,
        out_type=jax.ShapeDtypeStruct((FLAT,), f32),
        mesh=_mesh,
        compiler_params=_sc_params,
        scratch_types=[pltpu.VMEM((FLAT_TILE,), f32)] * 3,
    )
    return kern(p2, onorm)


def _combine_scale_bias_body(p_hbm, n_hbm, b_hbm, o_hbm, av, bv, cv, bvv):
    g0 = _wid() * FLAT_TILE
    pltpu.sync_copy(p_hbm.at[0].at[pl.ds(g0, FLAT_TILE)], av)
    pltpu.sync_copy(p_hbm.at[1].at[pl.ds(g0, FLAT_TILE)], bv)
    pltpu.sync_copy(n_hbm.at[pl.ds(g0, FLAT_TILE)], cv)
    pltpu.sync_copy(b_hbm, bvv)
    bvec = bvv[...]

    def body(i, c):
        s = pl.ds(i * L, L)
        av[s] = (av[s] + bv[s]) * cv[s] + bvec
        return c

    lax.fori_loop(0, ROWS_W_TILE, body, 0)
    pltpu.sync_copy(av, o_hbm.at[pl.ds(g0, FLAT_TILE)])


def _combine_scale_bias(p3, inorm, b):
    kern = pl.kernel(
        _combine_scale_bias_body,
        out_type=jax.ShapeDtypeStruct((FLAT,), f32),
        mesh=_mesh,
        compiler_params=_sc_params,
        scratch_types=[
            pltpu.VMEM((FLAT_TILE,), f32),
            pltpu.VMEM((FLAT_TILE,), f32),
            pltpu.VMEM((FLAT_TILE,), f32),
            pltpu.VMEM((OUT_F,), f32),
        ],
    )
    return kern(p3, inorm, b)


def _mm_body(x_ref, w_ref, o_ref):
    o_ref[...] = jnp.dot(x_ref[...], w_ref[...], preferred_element_type=f32)


def _matmul(x, W):
    return pl.pallas_call(
        _mm_body,
        grid=(10,),
        in_specs=[
            pl.BlockSpec((N // 10, IN_F), lambda i: (i, 0)),
            pl.BlockSpec((IN_F, OUT_F), lambda i: (0, 0)),
        ],
        out_specs=pl.BlockSpec((N // 10, OUT_F), lambda i: (i, 0)),
        out_shape=jax.ShapeDtypeStruct((N, OUT_F), f32),
    )(x, W)


def kernel(x, edge_index, W, b):
    src = edge_index[0].astype(i32)
    dst = edge_index[1].astype(i32)
    pad = N + (jnp.arange(E_PAD - E, dtype=i32) % (N_PAD - N))
    src2 = jnp.concatenate([src, pad]).reshape(N_CHUNKS, 128)
    dst2 = jnp.concatenate([dst, pad]).reshape(N_CHUNKS, 128)

    y = _matmul(x, W)
    y_p = jnp.concatenate([y, jnp.zeros((N_PAD - N, OUT_F), f32)], axis=0)

    ones_in = jnp.ones((128, OUT_F), f32)
    zeros_in = jnp.zeros((ROWS_SC_TILE, OUT_F), f32)

    p1, dA, dB = _round_deg(y_p, src2, dst2, ones_in, zeros_in)
    t1f, onormf, inormf = _combine_norm(
        p1.reshape(NC, FLAT), dA.reshape(NC, FLAT), dB.reshape(NC, FLAT))

    p2 = _round(t1f.reshape(N_PAD, OUT_F), src2, dst2, zeros_in)
    t2f = _combine_scale(p2.reshape(NC, FLAT), onormf)

    p3 = _round(t2f.reshape(N_PAD, OUT_F), src2, dst2, zeros_in)
    rstf = _combine_scale_bias(p3.reshape(NC, FLAT), inormf, b)

    return rstf.reshape(N_PAD, OUT_F)[:N]


# speedup vs baseline: 26.1553x; 1.1351x over previous
import functools

import jax
import jax.numpy as jnp
from jax import lax
from jax.experimental import pallas as pl
from jax.experimental.pallas import tpu as pltpu
from jax.experimental.pallas import tpu_sc as plsc

N = 10000
E = 320000
IN_F = 128
OUT_F = 16

NC = 2
NS = 16
NW = NC * NS
L = 16

N_PAD = 10240
ROWS_SC_TILE = N_PAD // NS
ROWS_W_TILE = N_PAD // NW
FLAT = N_PAD * OUT_F
FLAT_TILE = FLAT // NW

CPT = 80
N_CHUNKS = NW * CPT
E_PAD = N_CHUNKS * 128

_mesh = plsc.VectorSubcoreMesh(core_axis_name="c", subcore_axis_name="s")
_sc_params = pltpu.CompilerParams(use_tc_tiling_on_sc=False)
f32 = jnp.float32
i32 = jnp.int32


def _wid():
    return lax.axis_index("s") * NC + lax.axis_index("c")


NBUF = 8
DLAG = 2


def _gather_ring(y_hbm, srcv, rowsv, gsems, ssems, fire, drain):
    def _buf(b):
        return rowsv.at[pl.ds(b * 128, 128)]

    def _gstart(j, b):
        return pltpu.async_copy(y_hbm.at[srcv.at[j]], _buf(b), gsems.at[b])

    for b in range(NBUF):
        _gstart(b, b)

    def outer(q, c):
        for b in range(NBUF):
            j = q * NBUF + b
            pltpu.make_async_copy(y_hbm.at[srcv.at[j]], _buf(b), gsems.at[b]).wait()
            fire(j, _buf(b), ssems.at[b])
            jd = j - DLAG
            bd = (b - DLAG) % NBUF

            @pl.when(jnp.logical_and(jd >= 0, jd + NBUF < CPT))
            def _():
                drain(jd, _buf(bd), ssems.at[bd])
                _gstart(jd + NBUF, bd)
        return c

    lax.fori_loop(0, CPT // NBUF, outer, 0)
    for b in range(NBUF):
        drain(CPT - NBUF + b, _buf(b), ssems.at[b])


def _round_body_deg(y_hbm, src_hbm, dst_hbm, ones_hbm, zeros_hbm,
                    p_hbm, da_hbm, db_hbm,
                    srcv, dstv, onesv, rowsv, stagev, accum, dega, degb,
                    gsems, ssems):
    cid = lax.axis_index("c")
    sid = lax.axis_index("s")
    wid = sid * NC + cid
    r0 = sid * ROWS_SC_TILE
    pltpu.sync_copy(zeros_hbm, stagev)
    pltpu.sync_copy(stagev, accum.at[pl.ds(r0, ROWS_SC_TILE)])
    pltpu.sync_copy(stagev, dega.at[pl.ds(r0, ROWS_SC_TILE)])
    pltpu.sync_copy(stagev, degb.at[pl.ds(r0, ROWS_SC_TILE)])
    pltpu.sync_copy(ones_hbm, onesv)
    c0 = wid * CPT
    pltpu.sync_copy(src_hbm.at[pl.ds(c0, CPT)], srcv)
    pltpu.sync_copy(dst_hbm.at[pl.ds(c0, CPT)], dstv)
    plsc.subcore_barrier()

    def fire(j, rows, ssem):
        pltpu.async_copy(rows, accum.at[dstv.at[j]], ssem, add=True)
        pltpu.async_copy(onesv, dega.at[srcv.at[j]], ssem, add=True)
        pltpu.async_copy(onesv, degb.at[dstv.at[j]], ssem, add=True)

    def drain(j, rows, ssem):
        pltpu.make_async_copy(rows, accum.at[dstv.at[j]], ssem).wait()
        pltpu.make_async_copy(onesv, dega.at[srcv.at[j]], ssem).wait()
        pltpu.make_async_copy(onesv, degb.at[dstv.at[j]], ssem).wait()

    _gather_ring(y_hbm, srcv, rowsv, gsems, ssems, fire, drain)
    plsc.subcore_barrier()
    pltpu.sync_copy(accum.at[pl.ds(r0, ROWS_SC_TILE)], stagev)
    pltpu.sync_copy(stagev, p_hbm.at[cid].at[pl.ds(r0, ROWS_SC_TILE)])
    pltpu.sync_copy(dega.at[pl.ds(r0, ROWS_SC_TILE)], stagev)
    pltpu.sync_copy(stagev, da_hbm.at[cid].at[pl.ds(r0, ROWS_SC_TILE)])
    pltpu.sync_copy(degb.at[pl.ds(r0, ROWS_SC_TILE)], stagev)
    pltpu.sync_copy(stagev, db_hbm.at[cid].at[pl.ds(r0, ROWS_SC_TILE)])


def _round_body(y_hbm, src_hbm, dst_hbm, zeros_hbm, p_hbm,
                srcv, dstv, rowsv, stagev, accum, gsems, ssems):
    cid = lax.axis_index("c")
    sid = lax.axis_index("s")
    wid = sid * NC + cid
    r0 = sid * ROWS_SC_TILE
    pltpu.sync_copy(zeros_hbm, stagev)
    pltpu.sync_copy(stagev, accum.at[pl.ds(r0, ROWS_SC_TILE)])
    c0 = wid * CPT
    pltpu.sync_copy(src_hbm.at[pl.ds(c0, CPT)], srcv)
    pltpu.sync_copy(dst_hbm.at[pl.ds(c0, CPT)], dstv)
    plsc.subcore_barrier()

    def fire(j, rows, ssem):
        pltpu.async_copy(rows, accum.at[dstv.at[j]], ssem, add=True)

    def drain(j, rows, ssem):
        pltpu.make_async_copy(rows, accum.at[dstv.at[j]], ssem).wait()

    _gather_ring(y_hbm, srcv, rowsv, gsems, ssems, fire, drain)
    plsc.subcore_barrier()
    pltpu.sync_copy(accum.at[pl.ds(r0, ROWS_SC_TILE)], stagev)
    pltpu.sync_copy(stagev, p_hbm.at[cid].at[pl.ds(r0, ROWS_SC_TILE)])


def _round_deg(y_p, src2, dst2, ones_in, zeros_in):
    kern = pl.kernel(
        _round_body_deg,
        out_type=[jax.ShapeDtypeStruct((NC, N_PAD, OUT_F), f32)] * 3,
        mesh=_mesh,
        compiler_params=_sc_params,
        scratch_types=[
            pltpu.VMEM((CPT, 128), i32),
            pltpu.VMEM((CPT, 128), i32),
            pltpu.VMEM((128, OUT_F), f32),
            pltpu.VMEM((NBUF * 128, OUT_F), f32),
            pltpu.VMEM((ROWS_SC_TILE, OUT_F), f32),
            pltpu.VMEM_SHARED((N_PAD, OUT_F), f32),
            pltpu.VMEM_SHARED((N_PAD, OUT_F), f32),
            pltpu.VMEM_SHARED((N_PAD, OUT_F), f32),
            pltpu.SemaphoreType.DMA((NBUF,)),
            pltpu.SemaphoreType.DMA((NBUF,)),
        ],
    )
    return kern(y_p, src2, dst2, ones_in, zeros_in)


def _round(t_in, src2, dst2, zeros_in):
    kern = pl.kernel(
        _round_body,
        out_type=jax.ShapeDtypeStruct((NC, N_PAD, OUT_F), f32),
        mesh=_mesh,
        compiler_params=_sc_params,
        scratch_types=[
            pltpu.VMEM((CPT, 128), i32),
            pltpu.VMEM((CPT, 128), i32),
            pltpu.VMEM((NBUF * 128, OUT_F), f32),
            pltpu.VMEM((ROWS_SC_TILE, OUT_F), f32),
            pltpu.VMEM_SHARED((N_PAD, OUT_F), f32),
            pltpu.SemaphoreType.DMA((NBUF,)),
            pltpu.SemaphoreType.DMA((NBUF,)),
        ],
    )
    return kern(t_in, src2, dst2, zeros_in)


def _nrsqrt(dd):
    ii = lax.bitcast_convert_type(dd, i32)
    ii = 0x5F3759DF - (ii >> 1)
    yv = lax.bitcast_convert_type(ii, f32)
    yv = yv * (1.5 - 0.5 * dd * yv * yv)
    yv = yv * (1.5 - 0.5 * dd * yv * yv)
    yv = yv * (1.5 - 0.5 * dd * yv * yv)
    return yv


def _combine_norm_body(p_hbm, da_hbm, db_hbm, t1_hbm, on_hbm, in_hbm, av, bv, ov):
    g0 = _wid() * FLAT_TILE

    def _load2(src):
        pltpu.sync_copy(src.at[0].at[pl.ds(g0, FLAT_TILE)], av)
        pltpu.sync_copy(src.at[1].at[pl.ds(g0, FLAT_TILE)], bv)

    _load2(p_hbm)

    def add_body(i, c):
        s = pl.ds(i * L, L)
        ov[s] = av[s] + bv[s]
        return c

    lax.fori_loop(0, ROWS_W_TILE, add_body, 0)
    pltpu.sync_copy(ov, t1_hbm.at[pl.ds(g0, FLAT_TILE)])

    def norm_body(i, c):
        s = pl.ds(i * L, L)
        ov[s] = _nrsqrt(jnp.maximum(av[s] + bv[s], 1.0))
        return c

    _load2(da_hbm)
    lax.fori_loop(0, ROWS_W_TILE, norm_body, 0)
    pltpu.sync_copy(ov, on_hbm.at[pl.ds(g0, FLAT_TILE)])

    _load2(db_hbm)
    lax.fori_loop(0, ROWS_W_TILE, norm_body, 0)
    pltpu.sync_copy(ov, in_hbm.at[pl.ds(g0, FLAT_TILE)])


def _combine_norm(p1, dA, dB):
    kern = pl.kernel(
        _combine_norm_body,
        out_type=[jax.ShapeDtypeStruct((FLAT,), f32)] * 3,
        mesh=_mesh,
        compiler_params=_sc_params,
        scratch_types=[pltpu.VMEM((FLAT_TILE,), f32)] * 3,
    )
    return kern(p1, dA, dB)


def _combine_scale_body(p_hbm, n_hbm, o_hbm, av, bv, cv):
    g0 = _wid() * FLAT_TILE
    pltpu.sync_copy(p_hbm.at[0].at[pl.ds(g0, FLAT_TILE)], av)
    pltpu.sync_copy(p_hbm.at[1].at[pl.ds(g0, FLAT_TILE)], bv)
    pltpu.sync_copy(n_hbm.at[pl.ds(g0, FLAT_TILE)], cv)

    def body(i, c):
        s = pl.ds(i * L, L)
        av[s] = (av[s] + bv[s]) * cv[s]
        return c

    lax.fori_loop(0, ROWS_W_TILE, body, 0)
    pltpu.sync_copy(av, o_hbm.at[pl.ds(g0, FLAT_TILE)])


def _combine_scale(p2, onorm):
    kern = pl.kernel(
        _combine_scale_body,
        out_type=jax.ShapeDtypeStruct((FLAT,), f32),
        mesh=_mesh,
        compiler_params=_sc_params,
        scratch_types=[pltpu.VMEM((FLAT_TILE,), f32)] * 3,
    )
    return kern(p2, onorm)


def _combine_scale_bias_body(p_hbm, n_hbm, b_hbm, o_hbm, av, bv, cv, bvv):
    g0 = _wid() * FLAT_TILE
    pltpu.sync_copy(p_hbm.at[0].at[pl.ds(g0, FLAT_TILE)], av)
    pltpu.sync_copy(p_hbm.at[1].at[pl.ds(g0, FLAT_TILE)], bv)
    pltpu.sync_copy(n_hbm.at[pl.ds(g0, FLAT_TILE)], cv)
    pltpu.sync_copy(b_hbm, bvv)
    bvec = bvv[...]

    def body(i, c):
        s = pl.ds(i * L, L)
        av[s] = (av[s] + bv[s]) * cv[s] + bvec
        return c

    lax.fori_loop(0, ROWS_W_TILE, body, 0)
    pltpu.sync_copy(av, o_hbm.at[pl.ds(g0, FLAT_TILE)])


def _combine_scale_bias(p3, inorm, b):
    kern = pl.kernel(
        _combine_scale_bias_body,
        out_type=jax.ShapeDtypeStruct((FLAT,), f32),
        mesh=_mesh,
        compiler_params=_sc_params,
        scratch_types=[
            pltpu.VMEM((FLAT_TILE,), f32),
            pltpu.VMEM((FLAT_TILE,), f32),
            pltpu.VMEM((FLAT_TILE,), f32),
            pltpu.VMEM((OUT_F,), f32),
        ],
    )
    return kern(p3, inorm, b)


def _mm_body(x_ref, w_ref, o_ref):
    o_ref[...] = jnp.dot(x_ref[...], w_ref[...], preferred_element_type=f32)


def _matmul(x, W):
    return pl.pallas_call(
        _mm_body,
        grid=(10,),
        in_specs=[
            pl.BlockSpec((N // 10, IN_F), lambda i: (i, 0)),
            pl.BlockSpec((IN_F, OUT_F), lambda i: (0, 0)),
        ],
        out_specs=pl.BlockSpec((N // 10, OUT_F), lambda i: (i, 0)),
        out_shape=jax.ShapeDtypeStruct((N, OUT_F), f32),
    )(x, W)


def kernel(x, edge_index, W, b):
    src = edge_index[0].astype(i32)
    dst = edge_index[1].astype(i32)
    pad = N + (jnp.arange(E_PAD - E, dtype=i32) % (N_PAD - N))
    src2 = jnp.concatenate([src, pad]).reshape(N_CHUNKS, 128)
    dst2 = jnp.concatenate([dst, pad]).reshape(N_CHUNKS, 128)

    y = _matmul(x, W)
    y_p = jnp.concatenate([y, jnp.zeros((N_PAD - N, OUT_F), f32)], axis=0)

    ones_in = jnp.ones((128, OUT_F), f32)
    zeros_in = jnp.zeros((ROWS_SC_TILE, OUT_F), f32)

    p1, dA, dB = _round_deg(y_p, src2, dst2, ones_in, zeros_in)
    t1f, onormf, inormf = _combine_norm(
        p1.reshape(NC, FLAT), dA.reshape(NC, FLAT), dB.reshape(NC, FLAT))

    p2 = _round(t1f.reshape(N_PAD, OUT_F), src2, dst2, zeros_in)
    t2f = _combine_scale(p2.reshape(NC, FLAT), onormf)

    p3 = _round(t2f.reshape(N_PAD, OUT_F), src2, dst2, zeros_in)
    rstf = _combine_scale_bias(p3.reshape(NC, FLAT), inormf, b)

    return rstf.reshape(N_PAD, OUT_F)[:N]


# speedup vs baseline: 30.2534x; 1.1567x over previous
import functools

import jax
import jax.numpy as jnp
from jax import lax
from jax.experimental import pallas as pl
from jax.experimental.pallas import tpu as pltpu
from jax.experimental.pallas import tpu_sc as plsc

N = 10000
E = 320000
IN_F = 128
OUT_F = 16

NC = 2
NS = 16
NW = NC * NS
L = 16

N_PAD = 10240
ROWS_SC_TILE = N_PAD // NS
ROWS_W_TILE = N_PAD // NW
FLAT = N_PAD * OUT_F
FLAT_TILE = FLAT // NW

CPT = 80
N_CHUNKS = NW * CPT
E_PAD = N_CHUNKS * 128

_mesh = plsc.VectorSubcoreMesh(core_axis_name="c", subcore_axis_name="s")
_sc_params = pltpu.CompilerParams(use_tc_tiling_on_sc=False)
f32 = jnp.float32
i32 = jnp.int32


def _wid():
    return lax.axis_index("s") * NC + lax.axis_index("c")


NBUF = 8
DLAG = 2


def _gather_ring(y_hbm, srcv, rowsv, gsems, ssems, fire, drain):
    def _buf(b):
        return rowsv.at[pl.ds(b * 128, 128)]

    def _gstart(j, b):
        return pltpu.async_copy(y_hbm.at[srcv.at[j]], _buf(b), gsems.at[b])

    for b in range(NBUF):
        _gstart(b, b)

    def outer(q, c):
        for b in range(NBUF):
            j = q * NBUF + b
            pltpu.make_async_copy(y_hbm.at[srcv.at[j]], _buf(b), gsems.at[b]).wait()
            fire(j, _buf(b), ssems.at[b])
            jd = j - DLAG
            bd = (b - DLAG) % NBUF

            @pl.when(jnp.logical_and(jd >= 0, jd + NBUF < CPT))
            def _():
                drain(jd, _buf(bd), ssems.at[bd])
                _gstart(jd + NBUF, bd)
        return c

    lax.fori_loop(0, CPT // NBUF, outer, 0)
    for b in range(NBUF):
        drain(CPT - NBUF + b, _buf(b), ssems.at[b])


def _round_body_deg(y_hbm, src_hbm, dst_hbm, ones_hbm, zeros_hbm,
                    p_hbm, da_hbm, db_hbm,
                    srcv, dstv, onesv, rowsv, stagev, accum, dega, degb,
                    gsems, ssems):
    cid = lax.axis_index("c")
    sid = lax.axis_index("s")
    wid = sid * NC + cid
    r0 = sid * ROWS_SC_TILE
    pltpu.sync_copy(zeros_hbm, stagev)
    pltpu.sync_copy(stagev, accum.at[pl.ds(r0, ROWS_SC_TILE)])
    pltpu.sync_copy(stagev, dega.at[pl.ds(r0, ROWS_SC_TILE)])
    pltpu.sync_copy(stagev, degb.at[pl.ds(r0, ROWS_SC_TILE)])
    pltpu.sync_copy(ones_hbm, onesv)
    c0 = wid * CPT
    pltpu.sync_copy(src_hbm.at[pl.ds(c0, CPT)], srcv)
    pltpu.sync_copy(dst_hbm.at[pl.ds(c0, CPT)], dstv)
    plsc.subcore_barrier()

    def fire(j, rows, ssem):
        pltpu.async_copy(rows, accum.at[dstv.at[j]], ssem, add=True)
        pltpu.async_copy(onesv, dega.at[srcv.at[j]], ssem, add=True)
        pltpu.async_copy(onesv, degb.at[dstv.at[j]], ssem, add=True)

    def drain(j, rows, ssem):
        pltpu.make_async_copy(rows, accum.at[dstv.at[j]], ssem).wait()
        pltpu.make_async_copy(onesv, dega.at[srcv.at[j]], ssem).wait()
        pltpu.make_async_copy(onesv, degb.at[dstv.at[j]], ssem).wait()

    _gather_ring(y_hbm, srcv, rowsv, gsems, ssems, fire, drain)
    plsc.subcore_barrier()
    pltpu.sync_copy(accum.at[pl.ds(r0, ROWS_SC_TILE)], stagev)
    pltpu.sync_copy(stagev, p_hbm.at[cid].at[pl.ds(r0, ROWS_SC_TILE)])
    pltpu.sync_copy(dega.at[pl.ds(r0, ROWS_SC_TILE)], stagev)
    pltpu.sync_copy(stagev, da_hbm.at[cid].at[pl.ds(r0, ROWS_SC_TILE)])
    pltpu.sync_copy(degb.at[pl.ds(r0, ROWS_SC_TILE)], stagev)
    pltpu.sync_copy(stagev, db_hbm.at[cid].at[pl.ds(r0, ROWS_SC_TILE)])


def _round_body(y_hbm, src_hbm, dst_hbm, zeros_hbm, p_hbm,
                srcv, dstv, rowsv, stagev, accum, gsems, ssems):
    cid = lax.axis_index("c")
    sid = lax.axis_index("s")
    wid = sid * NC + cid
    r0 = sid * ROWS_SC_TILE
    pltpu.sync_copy(zeros_hbm, stagev)
    pltpu.sync_copy(stagev, accum.at[pl.ds(r0, ROWS_SC_TILE)])
    c0 = wid * CPT
    pltpu.sync_copy(src_hbm.at[pl.ds(c0, CPT)], srcv)
    pltpu.sync_copy(dst_hbm.at[pl.ds(c0, CPT)], dstv)
    plsc.subcore_barrier()

    def fire(j, rows, ssem):
        pltpu.async_copy(rows, accum.at[dstv.at[j]], ssem, add=True)

    def drain(j, rows, ssem):
        pltpu.make_async_copy(rows, accum.at[dstv.at[j]], ssem).wait()

    _gather_ring(y_hbm, srcv, rowsv, gsems, ssems, fire, drain)
    plsc.subcore_barrier()
    pltpu.sync_copy(accum.at[pl.ds(r0, ROWS_SC_TILE)], stagev)
    pltpu.sync_copy(stagev, p_hbm.at[cid].at[pl.ds(r0, ROWS_SC_TILE)])


def _round_deg(y_p, src2, dst2, ones_in, zeros_in):
    kern = pl.kernel(
        _round_body_deg,
        out_type=[jax.ShapeDtypeStruct((NC, N_PAD, OUT_F), f32)] * 3,
        mesh=_mesh,
        compiler_params=_sc_params,
        scratch_types=[
            pltpu.VMEM((CPT, 128), i32),
            pltpu.VMEM((CPT, 128), i32),
            pltpu.VMEM((128, OUT_F), f32),
            pltpu.VMEM((NBUF * 128, OUT_F), f32),
            pltpu.VMEM((ROWS_SC_TILE, OUT_F), f32),
            pltpu.VMEM_SHARED((N_PAD, OUT_F), f32),
            pltpu.VMEM_SHARED((N_PAD, OUT_F), f32),
            pltpu.VMEM_SHARED((N_PAD, OUT_F), f32),
            pltpu.SemaphoreType.DMA((NBUF,)),
            pltpu.SemaphoreType.DMA((NBUF,)),
        ],
    )
    return kern(y_p, src2, dst2, ones_in, zeros_in)


def _round(t_in, src2, dst2, zeros_in):
    kern = pl.kernel(
        _round_body,
        out_type=jax.ShapeDtypeStruct((NC, N_PAD, OUT_F), f32),
        mesh=_mesh,
        compiler_params=_sc_params,
        scratch_types=[
            pltpu.VMEM((CPT, 128), i32),
            pltpu.VMEM((CPT, 128), i32),
            pltpu.VMEM((NBUF * 128, OUT_F), f32),
            pltpu.VMEM((ROWS_SC_TILE, OUT_F), f32),
            pltpu.VMEM_SHARED((N_PAD, OUT_F), f32),
            pltpu.SemaphoreType.DMA((NBUF,)),
            pltpu.SemaphoreType.DMA((NBUF,)),
        ],
    )
    return kern(t_in, src2, dst2, zeros_in)


TCR = FLAT // 128


def _tc_combine_norm_body(p_ref, da_ref, db_ref, t1_ref, on_ref, in_ref):
    t1_ref[...] = p_ref[0] + p_ref[1]
    on_ref[...] = lax.rsqrt(jnp.maximum(da_ref[0] + da_ref[1], 1.0))
    in_ref[...] = lax.rsqrt(jnp.maximum(db_ref[0] + db_ref[1], 1.0))


def _combine_norm(p1, dA, dB):
    return pl.pallas_call(
        _tc_combine_norm_body,
        out_shape=[jax.ShapeDtypeStruct((TCR, 128), f32)] * 3,
    )(p1, dA, dB)


def _tc_combine_scale_body(p_ref, n_ref, o_ref):
    o_ref[...] = (p_ref[0] + p_ref[1]) * n_ref[...]


def _combine_scale(p2, onorm):
    return pl.pallas_call(
        _tc_combine_scale_body,
        out_shape=jax.ShapeDtypeStruct((TCR, 128), f32),
    )(p2, onorm)


def _tc_combine_scale_bias_body(p_ref, n_ref, b_ref, o_ref):
    o_ref[...] = (p_ref[0] + p_ref[1]) * n_ref[...] + b_ref[...]


def _combine_scale_bias(p3, inorm, b_tile):
    return pl.pallas_call(
        _tc_combine_scale_bias_body,
        out_shape=jax.ShapeDtypeStruct((TCR, 128), f32),
    )(p3, inorm, b_tile)


def _mm_body(x_ref, w_ref, o_ref):
    o_ref[...] = jnp.dot(x_ref[...], w_ref[...], preferred_element_type=f32)


def _matmul(x, W):
    return pl.pallas_call(
        _mm_body,
        grid=(10,),
        in_specs=[
            pl.BlockSpec((N // 10, IN_F), lambda i: (i, 0)),
            pl.BlockSpec((IN_F, OUT_F), lambda i: (0, 0)),
        ],
        out_specs=pl.BlockSpec((N // 10, OUT_F), lambda i: (i, 0)),
        out_shape=jax.ShapeDtypeStruct((N, OUT_F), f32),
    )(x, W)


def kernel(x, edge_index, W, b):
    src = edge_index[0].astype(i32)
    dst = edge_index[1].astype(i32)
    pad = N + (jnp.arange(E_PAD - E, dtype=i32) % (N_PAD - N))
    src2 = jnp.concatenate([src, pad]).reshape(N_CHUNKS, 128)
    dst2 = jnp.concatenate([dst, pad]).reshape(N_CHUNKS, 128)

    y = _matmul(x, W)
    y_p = jnp.concatenate([y, jnp.zeros((N_PAD - N, OUT_F), f32)], axis=0)

    ones_in = jnp.ones((128, OUT_F), f32)
    zeros_in = jnp.zeros((ROWS_SC_TILE, OUT_F), f32)
    b_tile = jnp.tile(b, 128 // OUT_F)

    p1, dA, dB = _round_deg(y_p, src2, dst2, ones_in, zeros_in)
    t1, onorm, inorm = _combine_norm(
        p1.reshape(NC, TCR, 128), dA.reshape(NC, TCR, 128), dB.reshape(NC, TCR, 128))

    p2 = _round(t1.reshape(N_PAD, OUT_F), src2, dst2, zeros_in)
    t2 = _combine_scale(p2.reshape(NC, TCR, 128), onorm)

    p3 = _round(t2.reshape(N_PAD, OUT_F), src2, dst2, zeros_in)
    rst = _combine_scale_bias(p3.reshape(NC, TCR, 128), inorm, b_tile)

    return rst.reshape(N_PAD, OUT_F)[:N]


# speedup vs baseline: 32.0947x; 1.0609x over previous
import functools

import jax
import jax.numpy as jnp
from jax import lax
from jax.experimental import pallas as pl
from jax.experimental.pallas import tpu as pltpu
from jax.experimental.pallas import tpu_sc as plsc

N = 10000
E = 320000
IN_F = 128
OUT_F = 16

NC = 2
NS = 16
NW = NC * NS
L = 16

N_PAD = 10240
ROWS_SC_TILE = N_PAD // NS
ROWS_W_TILE = N_PAD // NW
FLAT = N_PAD * OUT_F
FLAT_TILE = FLAT // NW

CPT = 80
N_CHUNKS = NW * CPT
E_PAD = N_CHUNKS * 128

_mesh = plsc.VectorSubcoreMesh(core_axis_name="c", subcore_axis_name="s")
_sc_params = pltpu.CompilerParams(use_tc_tiling_on_sc=False)
f32 = jnp.float32
i32 = jnp.int32


def _wid():
    return lax.axis_index("s") * NC + lax.axis_index("c")


NBUF = 8
DLAG = 2


def _gather_ring(y_hbm, srcv, rowsv, gsems, ssems, fire, drain):
    def _buf(b):
        return rowsv.at[pl.ds(b * 128, 128)]

    def _gstart(j, b):
        return pltpu.async_copy(y_hbm.at[srcv.at[j]], _buf(b), gsems.at[b])

    for b in range(NBUF):
        _gstart(b, b)

    def outer(q, c):
        for b in range(NBUF):
            j = q * NBUF + b
            pltpu.make_async_copy(y_hbm.at[srcv.at[j]], _buf(b), gsems.at[b]).wait()
            fire(j, _buf(b), ssems.at[b])
            jd = j - DLAG
            bd = (b - DLAG) % NBUF

            @pl.when(jnp.logical_and(jd >= 0, jd + NBUF < CPT))
            def _():
                drain(jd, _buf(bd), ssems.at[bd])
                _gstart(jd + NBUF, bd)
        return c

    lax.fori_loop(0, CPT // NBUF, outer, 0)
    for b in range(NBUF):
        drain(CPT - NBUF + b, _buf(b), ssems.at[b])


def _round_body_deg(y_hbm, ed_hbm, ones_hbm, zeros_hbm,
                    p_hbm, da_hbm, db_hbm,
                    srcv, dstv, onesv, rowsv, stagev, accum, dega, degb,
                    gsems, ssems):
    cid = lax.axis_index("c")
    sid = lax.axis_index("s")
    wid = sid * NC + cid
    r0 = sid * ROWS_SC_TILE
    pltpu.sync_copy(zeros_hbm, stagev)
    pltpu.sync_copy(stagev, accum.at[pl.ds(r0, ROWS_SC_TILE)])
    pltpu.sync_copy(stagev, dega.at[pl.ds(r0, ROWS_SC_TILE)])
    pltpu.sync_copy(stagev, degb.at[pl.ds(r0, ROWS_SC_TILE)])
    pltpu.sync_copy(ones_hbm, onesv)
    c0 = wid * CPT
    pltpu.sync_copy(ed_hbm.at[0].at[pl.ds(c0, CPT)], srcv)
    pltpu.sync_copy(ed_hbm.at[1].at[pl.ds(c0, CPT)], dstv)
    plsc.subcore_barrier()

    def fire(j, rows, ssem):
        pltpu.async_copy(rows, accum.at[dstv.at[j]], ssem, add=True)
        pltpu.async_copy(onesv, dega.at[srcv.at[j]], ssem, add=True)
        pltpu.async_copy(onesv, degb.at[dstv.at[j]], ssem, add=True)

    def drain(j, rows, ssem):
        pltpu.make_async_copy(rows, accum.at[dstv.at[j]], ssem).wait()
        pltpu.make_async_copy(onesv, dega.at[srcv.at[j]], ssem).wait()
        pltpu.make_async_copy(onesv, degb.at[dstv.at[j]], ssem).wait()

    _gather_ring(y_hbm, srcv, rowsv, gsems, ssems, fire, drain)
    plsc.subcore_barrier()
    pltpu.sync_copy(accum.at[pl.ds(r0, ROWS_SC_TILE)], stagev)
    pltpu.sync_copy(stagev, p_hbm.at[cid].at[pl.ds(r0, ROWS_SC_TILE)])
    pltpu.sync_copy(dega.at[pl.ds(r0, ROWS_SC_TILE)], stagev)
    pltpu.sync_copy(stagev, da_hbm.at[cid].at[pl.ds(r0, ROWS_SC_TILE)])
    pltpu.sync_copy(degb.at[pl.ds(r0, ROWS_SC_TILE)], stagev)
    pltpu.sync_copy(stagev, db_hbm.at[cid].at[pl.ds(r0, ROWS_SC_TILE)])


def _round_body(y_hbm, ed_hbm, zeros_hbm, p_hbm,
                srcv, dstv, rowsv, stagev, accum, gsems, ssems):
    cid = lax.axis_index("c")
    sid = lax.axis_index("s")
    wid = sid * NC + cid
    r0 = sid * ROWS_SC_TILE
    pltpu.sync_copy(zeros_hbm, stagev)
    pltpu.sync_copy(stagev, accum.at[pl.ds(r0, ROWS_SC_TILE)])
    c0 = wid * CPT
    pltpu.sync_copy(ed_hbm.at[0].at[pl.ds(c0, CPT)], srcv)
    pltpu.sync_copy(ed_hbm.at[1].at[pl.ds(c0, CPT)], dstv)
    plsc.subcore_barrier()

    def fire(j, rows, ssem):
        pltpu.async_copy(rows, accum.at[dstv.at[j]], ssem, add=True)

    def drain(j, rows, ssem):
        pltpu.make_async_copy(rows, accum.at[dstv.at[j]], ssem).wait()

    _gather_ring(y_hbm, srcv, rowsv, gsems, ssems, fire, drain)
    plsc.subcore_barrier()
    pltpu.sync_copy(accum.at[pl.ds(r0, ROWS_SC_TILE)], stagev)
    pltpu.sync_copy(stagev, p_hbm.at[cid].at[pl.ds(r0, ROWS_SC_TILE)])


def _round_deg(y_p, ed, ones_in, zeros_in):
    kern = pl.kernel(
        _round_body_deg,
        out_type=[jax.ShapeDtypeStruct((NC, N_PAD, OUT_F), f32)] * 3,
        mesh=_mesh,
        compiler_params=_sc_params,
        scratch_types=[
            pltpu.VMEM((CPT, 128), i32),
            pltpu.VMEM((CPT, 128), i32),
            pltpu.VMEM((128, OUT_F), f32),
            pltpu.VMEM((NBUF * 128, OUT_F), f32),
            pltpu.VMEM((ROWS_SC_TILE, OUT_F), f32),
            pltpu.VMEM_SHARED((N_PAD, OUT_F), f32),
            pltpu.VMEM_SHARED((N_PAD, OUT_F), f32),
            pltpu.VMEM_SHARED((N_PAD, OUT_F), f32),
            pltpu.SemaphoreType.DMA((NBUF,)),
            pltpu.SemaphoreType.DMA((NBUF,)),
        ],
    )
    return kern(y_p, ed, ones_in, zeros_in)


def _round(t_in, ed, zeros_in):
    kern = pl.kernel(
        _round_body,
        out_type=jax.ShapeDtypeStruct((NC, N_PAD, OUT_F), f32),
        mesh=_mesh,
        compiler_params=_sc_params,
        scratch_types=[
            pltpu.VMEM((CPT, 128), i32),
            pltpu.VMEM((CPT, 128), i32),
            pltpu.VMEM((NBUF * 128, OUT_F), f32),
            pltpu.VMEM((ROWS_SC_TILE, OUT_F), f32),
            pltpu.VMEM_SHARED((N_PAD, OUT_F), f32),
            pltpu.SemaphoreType.DMA((NBUF,)),
            pltpu.SemaphoreType.DMA((NBUF,)),
        ],
    )
    return kern(t_in, ed, zeros_in)


TCR = FLAT // 128


def _tc_combine_norm_body(p_ref, da_ref, db_ref, t1_ref, on_ref, in_ref):
    t1_ref[...] = p_ref[0] + p_ref[1]
    on_ref[...] = lax.rsqrt(jnp.maximum(da_ref[0] + da_ref[1], 1.0))
    in_ref[...] = lax.rsqrt(jnp.maximum(db_ref[0] + db_ref[1], 1.0))


def _combine_norm(p1, dA, dB):
    return pl.pallas_call(
        _tc_combine_norm_body,
        out_shape=[jax.ShapeDtypeStruct((TCR, 128), f32)] * 3,
    )(p1, dA, dB)


def _tc_combine_scale_body(p_ref, n_ref, o_ref):
    o_ref[...] = (p_ref[0] + p_ref[1]) * n_ref[...]


def _combine_scale(p2, onorm):
    return pl.pallas_call(
        _tc_combine_scale_body,
        out_shape=jax.ShapeDtypeStruct((TCR, 128), f32),
    )(p2, onorm)


def _tc_combine_scale_bias_body(p_ref, n_ref, b_ref, o_ref):
    o_ref[...] = (p_ref[0] + p_ref[1]) * n_ref[...] + b_ref[...]


def _combine_scale_bias(p3, inorm, b_tile):
    return pl.pallas_call(
        _tc_combine_scale_bias_body,
        out_shape=jax.ShapeDtypeStruct((TCR, 128), f32),
    )(p3, inorm, b_tile)


def _mm_body(x_ref, w_ref, o_ref):
    o_ref[...] = jnp.dot(x_ref[...], w_ref[...], preferred_element_type=f32)


def _matmul(x, W):
    return pl.pallas_call(
        _mm_body,
        grid=(N_PAD // 1280,),
        in_specs=[
            pl.BlockSpec((1280, IN_F), lambda i: (i, 0)),
            pl.BlockSpec((IN_F, OUT_F), lambda i: (0, 0)),
        ],
        out_specs=pl.BlockSpec((1280, OUT_F), lambda i: (i, 0)),
        out_shape=jax.ShapeDtypeStruct((N_PAD, OUT_F), f32),
    )(x, W)


def kernel(x, edge_index, W, b):
    pad = N + (jnp.arange(E_PAD - E, dtype=i32) % (N_PAD - N))
    ed = jnp.concatenate(
        [edge_index.astype(i32), jnp.broadcast_to(pad, (2, E_PAD - E))], axis=1
    ).reshape(2, N_CHUNKS, 128)

    y_p = _matmul(x, W)

    ones_in = jnp.ones((128, OUT_F), f32)
    zeros_in = jnp.zeros((ROWS_SC_TILE, OUT_F), f32)
    b_tile = jnp.tile(b, 128 // OUT_F)

    p1, dA, dB = _round_deg(y_p, ed, ones_in, zeros_in)
    t1, onorm, inorm = _combine_norm(
        p1.reshape(NC, TCR, 128), dA.reshape(NC, TCR, 128), dB.reshape(NC, TCR, 128))

    p2 = _round(t1.reshape(N_PAD, OUT_F), ed, zeros_in)
    t2 = _combine_scale(p2.reshape(NC, TCR, 128), onorm)

    p3 = _round(t2.reshape(N_PAD, OUT_F), ed, zeros_in)
    rst = _combine_scale_bias(p3.reshape(NC, TCR, 128), inorm, b_tile)

    return rst.reshape(N_PAD, OUT_F)[:N]
